# Initial kernel scaffold; baseline (speedup 1.0000x reference)
import jax
import jax.numpy as jnp
from jax.experimental import pallas as pl


def kernel(x, map_table, min_vals):
    raise NotImplementedError("write your pallas kernel here")


# speedup vs baseline: 55.6445x; 55.6445x over previous
import functools

import jax
import jax.numpy as jnp
from jax import lax
from jax.experimental import pallas as pl
from jax.experimental.pallas import tpu as pltpu
from jax.experimental.pallas import tpu_sc as plsc

LANES = 16


def _tokenize_body(n_cat, per_w, n_cores,
                   x_hbm, tab_hbm, min_hbm, out_hbm,
                   x_v, tab_v, min_v, out_v):
    wid = lax.axis_index("s") * n_cores + lax.axis_index("c")
    base = wid * per_w
    pltpu.sync_copy(x_hbm.at[pl.ds(base, per_w)], x_v)
    pltpu.sync_copy(tab_hbm, tab_v)
    pltpu.sync_copy(min_hbm, min_v)

    iota = lax.iota(jnp.int32, LANES)
    n_vecs = per_w // LANES

    def body(p, carry):
        off = pl.multiple_of(p * LANES, LANES)
        c = lax.rem(off + iota, n_cat)
        m = plsc.load_gather(min_v, [c])
        v = x_v[pl.ds(off, LANES)] - m
        out_v[pl.ds(off, LANES)] = plsc.load_gather(tab_v, [c, v])
        return carry

    lax.fori_loop(0, n_vecs, body, 0)
    pltpu.sync_copy(out_v, out_hbm.at[pl.ds(base, per_w)])


def kernel(x, map_table, min_vals):
    batch, n_cat = x.shape
    n = batch * n_cat
    info = plsc.get_sparse_core_info()
    n_workers = info.num_cores * info.num_subcores
    per_w = n // n_workers
    assert per_w % LANES == 0 and per_w % n_cat == 0 and n == per_w * n_workers

    mesh = plsc.VectorSubcoreMesh(core_axis_name="c", subcore_axis_name="s")
    body = functools.partial(_tokenize_body, n_cat, per_w, info.num_cores)
    run = pl.kernel(
        body,
        out_type=jax.ShapeDtypeStruct((n,), jnp.int32),
        mesh=mesh,
        scratch_types=[
            pltpu.VMEM((per_w,), jnp.int32),
            pltpu.VMEM(map_table.shape, jnp.int32),
            pltpu.VMEM(min_vals.shape, jnp.int32),
            pltpu.VMEM((per_w,), jnp.int32),
        ],
        compiler_params=pltpu.CompilerParams(needs_layout_passes=False),
    )
    out = run(x.reshape(n), map_table, min_vals)
    return out.reshape(batch, n_cat)


# speedup vs baseline: 62.4576x; 1.1224x over previous
import functools
import math

import jax
import jax.numpy as jnp
from jax import lax
from jax.experimental import pallas as pl
from jax.experimental.pallas import tpu as pltpu
from jax.experimental.pallas import tpu_sc as plsc

LANES = 16


def _tokenize_body(n_cat, n_cols, per_w, period_v, n_cores,
                   x_hbm, tab_hbm, min_hbm, out_hbm,
                   x_v, tab_v, min_v, off_v, out_v):
    wid = lax.axis_index("s") * n_cores + lax.axis_index("c")
    base = wid * per_w
    pltpu.sync_copy(x_hbm.at[pl.ds(base, per_w)], x_v)
    pltpu.sync_copy(tab_hbm, tab_v)
    pltpu.sync_copy(min_hbm, min_v)

    iota = lax.iota(jnp.int32, LANES)
    for j in range(period_v):
        c = lax.rem(j * LANES + iota, n_cat)
        m = plsc.load_gather(min_v, [c])
        off_v[pl.ds(j * LANES, LANES)] = c * n_cols - m

    n_groups = per_w // (period_v * LANES)

    @plsc.parallel_loop(0, n_groups)
    def _(g):
        gbase = pl.multiple_of(g * (period_v * LANES), period_v * LANES)
        for j in range(period_v):
            s = gbase + j * LANES
            idx = x_v[pl.ds(s, LANES)] + off_v[pl.ds(j * LANES, LANES)]
            out_v[pl.ds(s, LANES)] = plsc.load_gather(tab_v, [idx])

    pltpu.sync_copy(out_v, out_hbm.at[pl.ds(base, per_w)])


def kernel(x, map_table, min_vals):
    batch, n_cat = x.shape
    n_cols = map_table.shape[1]
    n = batch * n_cat
    info = plsc.get_sparse_core_info()
    n_workers = info.num_cores * info.num_subcores
    per_w = n // n_workers
    period = n_cat * LANES // math.gcd(n_cat, LANES)
    period_v = period // LANES
    assert n == per_w * n_workers and per_w % period == 0

    mesh = plsc.VectorSubcoreMesh(core_axis_name="c", subcore_axis_name="s")
    body = functools.partial(_tokenize_body, n_cat, n_cols, per_w, period_v,
                             info.num_cores)
    run = pl.kernel(
        body,
        out_type=jax.ShapeDtypeStruct((n,), jnp.int32),
        mesh=mesh,
        scratch_types=[
            pltpu.VMEM((per_w,), jnp.int32),
            pltpu.VMEM((n_cat * n_cols,), jnp.int32),
            pltpu.VMEM((n_cat,), jnp.int32),
            pltpu.VMEM((period_v * LANES,), jnp.int32),
            pltpu.VMEM((per_w,), jnp.int32),
        ],
        compiler_params=pltpu.CompilerParams(needs_layout_passes=False),
    )
    out = run(x.reshape(n), map_table.reshape(n_cat * n_cols), min_vals)
    return out.reshape(batch, n_cat)


# speedup vs baseline: 87.4668x; 1.4004x over previous
import functools

import jax
import jax.numpy as jnp
from jax import lax
from jax.experimental import pallas as pl
from jax.experimental.pallas import tpu as pltpu
from jax.experimental.pallas import tpu_sc as plsc

LANES = 16


def _tokenize_body(n_cat, rows_w, rows_ch, n_cores,
                   x_hbm, tab_hbm, min_hbm, out_hbm,
                   x_v, tab_v, min_v, out_v):
    wid = lax.axis_index("s") * n_cores + lax.axis_index("c")
    base = wid * rows_w
    pltpu.sync_copy(tab_hbm, tab_v)
    pltpu.sync_copy(min_hbm, min_v)

    c0 = lax.iota(jnp.int32, LANES)
    c1 = c0 + (n_cat - LANES)
    m0 = plsc.load_gather(min_v, [c0])
    m1 = plsc.load_gather(min_v, [c1])
    s1 = n_cat - LANES

    for ch in range(rows_w // rows_ch):
        cbase = base + ch * rows_ch
        pltpu.sync_copy(x_hbm.at[pl.ds(cbase, rows_ch)], x_v)

        @plsc.parallel_loop(0, rows_ch)
        def _(r):
            v0 = x_v[r, pl.ds(0, LANES)] - m0
            out_v[r, pl.ds(0, LANES)] = plsc.load_gather(tab_v, [c0, v0])
            v1 = x_v[r, pl.ds(s1, LANES)] - m1
            out_v[r, pl.ds(s1, LANES)] = plsc.load_gather(tab_v, [c1, v1])

        pltpu.sync_copy(out_v, out_hbm.at[pl.ds(cbase, rows_ch)])


def kernel(x, map_table, min_vals):
    batch, n_cat = x.shape
    assert LANES < n_cat <= 2 * LANES
    info = plsc.get_sparse_core_info()
    n_workers = info.num_cores * info.num_subcores
    rows_w = batch // n_workers
    rows_ch = min(rows_w, 256)
    assert batch == rows_w * n_workers and rows_w % rows_ch == 0

    mesh = plsc.VectorSubcoreMesh(core_axis_name="c", subcore_axis_name="s")
    body = functools.partial(_tokenize_body, n_cat, rows_w, rows_ch,
                             info.num_cores)
    run = pl.kernel(
        body,
        out_type=jax.ShapeDtypeStruct((batch, n_cat), jnp.int32),
        mesh=mesh,
        scratch_types=[
            pltpu.VMEM((rows_ch, n_cat), jnp.int32),
            pltpu.VMEM(map_table.shape, jnp.int32),
            pltpu.VMEM(min_vals.shape, jnp.int32),
            pltpu.VMEM((rows_ch, n_cat), jnp.int32),
        ],
        compiler_params=pltpu.CompilerParams(needs_layout_passes=False),
    )
    return run(x, map_table, min_vals)


# speedup vs baseline: 87.5508x; 1.0010x over previous
import functools

import jax
import jax.numpy as jnp
from jax import lax
from jax.experimental import pallas as pl
from jax.experimental.pallas import tpu as pltpu
from jax.experimental.pallas import tpu_sc as plsc

LANES = 16


def _tokenize_body(n_cat, rows_w, rows_ch, n_cores,
                   x_hbm, tab_hbm, min_hbm, out_hbm,
                   x_v, tab_v, min_v, out_v):
    wid = lax.axis_index("s") * n_cores + lax.axis_index("c")
    base = wid * rows_w
    pltpu.sync_copy(tab_hbm, tab_v)
    pltpu.sync_copy(min_hbm, min_v)

    c0 = lax.iota(jnp.int32, LANES)
    c1 = c0 + (n_cat - LANES)
    m0 = plsc.load_gather(min_v, [c0])
    m1 = plsc.load_gather(min_v, [c1])
    s1 = n_cat - LANES

    for ch in range(rows_w // rows_ch):
        cbase = base + ch * rows_ch
        pltpu.sync_copy(x_hbm.at[pl.ds(cbase, rows_ch)], x_v)

        @plsc.parallel_loop(0, rows_ch)
        def _(r):
            v0 = x_v[r, pl.ds(0, LANES)] - m0
            out_v[r, pl.ds(0, LANES)] = plsc.load_gather(tab_v, [c0, v0])
            v1 = x_v[r, pl.ds(s1, LANES)] - m1
            out_v[r, pl.ds(s1, LANES)] = plsc.load_gather(tab_v, [c1, v1])

        pltpu.sync_copy(out_v, out_hbm.at[pl.ds(cbase, rows_ch)])


def kernel(x, map_table, min_vals):
    batch, n_cat = x.shape
    assert LANES < n_cat <= 2 * LANES
    info = plsc.get_sparse_core_info()
    n_workers = info.num_cores * info.num_subcores
    rows_w = batch // n_workers
    rows_ch = min(rows_w, 256)
    assert batch == rows_w * n_workers and rows_w % rows_ch == 0

    mesh = plsc.VectorSubcoreMesh(core_axis_name="c", subcore_axis_name="s")
    body = functools.partial(_tokenize_body, n_cat, rows_w, rows_ch,
                             info.num_cores)
    run = pl.kernel(
        body,
        out_type=jax.ShapeDtypeStruct((batch, n_cat), jnp.int32),
        mesh=mesh,
        scratch_types=[
            pltpu.VMEM((rows_ch, n_cat), jnp.int32),
            pltpu.VMEM(map_table.shape, jnp.int32),
            pltpu.VMEM(min_vals.shape, jnp.int32),
            pltpu.VMEM((rows_ch, n_cat), jnp.int32),
        ],
        compiler_params=pltpu.CompilerParams(needs_layout_passes=False,
                                             use_tc_tiling_on_sc=True),
    )
    return run(x, map_table, min_vals)


# speedup vs baseline: 89.6290x; 1.0237x over previous
import functools

import jax
import jax.numpy as jnp
from jax import lax
from jax.experimental import pallas as pl
from jax.experimental.pallas import tpu as pltpu
from jax.experimental.pallas import tpu_sc as plsc

LANES = 16
NBUF = 2


def _tokenize_body(n_cat, rows_w, rows_ch, n_cores,
                   x_hbm, tab_hbm, min_hbm, out_hbm,
                   x_bufs, out_bufs, tab_v, min_v, in_sems, out_sems):
    wid = lax.axis_index("s") * n_cores + lax.axis_index("c")
    base = wid * rows_w
    pltpu.sync_copy(tab_hbm, tab_v)
    pltpu.sync_copy(min_hbm, min_v)

    n_ch = rows_w // rows_ch

    def in_copy(ch, b):
        return pltpu.make_async_copy(
            x_hbm.at[pl.ds(base + ch * rows_ch, rows_ch)], x_bufs[b],
            in_sems[b])

    def out_copy(ch, b):
        return pltpu.make_async_copy(
            out_bufs[b], out_hbm.at[pl.ds(base + ch * rows_ch, rows_ch)],
            out_sems[b])

    c0 = lax.iota(jnp.int32, LANES)
    c1 = c0 + (n_cat - LANES)
    m0 = plsc.load_gather(min_v, [c0])
    m1 = plsc.load_gather(min_v, [c1])
    s1 = n_cat - LANES

    in_copy(0, 0).start()
    for ch in range(n_ch):
        b = ch % NBUF
        in_copy(ch, b).wait()
        if ch + 1 < n_ch:
            in_copy(ch + 1, (ch + 1) % NBUF).start()
        if ch >= NBUF:
            out_copy(ch - NBUF, b).wait()
        x_v = x_bufs[b]
        out_v = out_bufs[b]

        @plsc.parallel_loop(0, rows_ch)
        def _(r):
            v0 = x_v[r, pl.ds(0, LANES)] - m0
            out_v[r, pl.ds(0, LANES)] = plsc.load_gather(tab_v, [c0, v0])
            v1 = x_v[r, pl.ds(s1, LANES)] - m1
            out_v[r, pl.ds(s1, LANES)] = plsc.load_gather(tab_v, [c1, v1])

        out_copy(ch, b).start()
    for ch in range(max(n_ch - NBUF, 0), n_ch):
        out_copy(ch, ch % NBUF).wait()


def kernel(x, map_table, min_vals):
    batch, n_cat = x.shape
    assert LANES < n_cat <= 2 * LANES
    info = plsc.get_sparse_core_info()
    n_workers = info.num_cores * info.num_subcores
    rows_w = batch // n_workers
    rows_ch = min(rows_w, 128)
    assert batch == rows_w * n_workers and rows_w % rows_ch == 0
    assert rows_w // rows_ch >= NBUF

    mesh = plsc.VectorSubcoreMesh(core_axis_name="c", subcore_axis_name="s")
    body = functools.partial(_tokenize_body, n_cat, rows_w, rows_ch,
                             info.num_cores)
    run = pl.kernel(
        body,
        out_type=jax.ShapeDtypeStruct((batch, n_cat), jnp.int32),
        mesh=mesh,
        scratch_types=[
            [pltpu.VMEM((rows_ch, n_cat), jnp.int32) for _ in range(NBUF)],
            [pltpu.VMEM((rows_ch, n_cat), jnp.int32) for _ in range(NBUF)],
            pltpu.VMEM(map_table.shape, jnp.int32),
            pltpu.VMEM(min_vals.shape, jnp.int32),
            [pltpu.SemaphoreType.DMA for _ in range(NBUF)],
            [pltpu.SemaphoreType.DMA for _ in range(NBUF)],
        ],
        compiler_params=pltpu.CompilerParams(needs_layout_passes=False,
                                             skip_device_barrier=True),
    )
    return run(x, map_table, min_vals)


# speedup vs baseline: 640.0802x; 7.1414x over previous
import jax
import jax.numpy as jnp
from jax.experimental import pallas as pl
from jax.experimental.pallas import tpu as pltpu


def _tok_body(xT_ref, tab_ref, min_ref, outT_ref):
    v = xT_ref[...] - min_ref[...]
    outT_ref[...] = jnp.take_along_axis(
        tab_ref[...], v, axis=1, mode="promise_in_bounds")


def kernel(x, map_table, min_vals):
    batch, n_cat = x.shape
    run = pl.pallas_call(
        _tok_body,
        out_shape=jax.ShapeDtypeStruct((n_cat, batch), jnp.int32),
    )
    outT = run(x.T, map_table, min_vals[:, None])
    return outT.T
